# baseline (speedup 1.0000x reference)
import functools

import jax
import jax.numpy as jnp
from jax import lax
from jax.experimental import pallas as pl
from jax.experimental.pallas import tpu as pltpu
from jax.experimental.pallas import tpu_sc as plsc

Z, H, W = 14, 32, 64
C = 70
NC, NS, LANES = 2, 16, 16
CSTARTS = (0, 16, 32, 48, C - LANES)


CNT_Z = tuple(min(z + 1, Z - 2) - max(z - 1, 1) + 1 for z in range(Z))
ZP = 2
NPIECES = Z // ZP
WUNROLL = 8


def _fold_body(x_hbm, out_hbm, bi0, bi1, bo0, bo1, si0, si1, so0, so1):
    cid = lax.axis_index("c")
    sid = lax.axis_index("s")
    h = sid * NC + cid

    ch = jnp.minimum(h + 1, H - 2) - jnp.maximum(h - 1, 1) + 1
    chv = jnp.full((LANES,), ch).astype(jnp.float32)

    bufs_in = (bi0, bi1)
    bufs_out = (bo0, bo1)
    sems_in = (si0, si1)
    sems_out = (so0, so1)

    def stage_in(i):
        return pltpu.async_copy(
            x_hbm.at[0, pl.ds(i * ZP, ZP), h, :, :], bufs_in[i % 2], sems_in[i % 2]
        )

    in_flight = [stage_in(0), stage_in(1)]
    out_flight = [None, None]
    for i in range(NPIECES):
        b = i % 2
        in_flight[b].wait()
        if i + 2 < NPIECES:
            in_flight[b] = stage_in(i + 2)
        if out_flight[b] is not None:
            out_flight[b].wait()
        src, dst = bufs_in[b], bufs_out[b]
        for zi in range(ZP):
            z = i * ZP + zi
            cnt = chv * float(CNT_Z[z] * 3)
            rcp = 1.0 / cnt
            def w_iter(wi, carry, zi=zi, cnt=cnt, rcp=rcp, src=src, dst=dst):
                for wu in range(WUNROLL):
                    w = wi * WUNROLL + wu
                    for c0 in CSTARTS:
                        sl = pl.ds(c0, LANES)
                        acc = src[zi, w, sl] * cnt
                        dst[zi, w, sl] = acc * rcp
                return carry
            lax.fori_loop(0, W // WUNROLL, w_iter, 0)
        out_flight[b] = pltpu.async_copy(
            dst, out_hbm.at[0, pl.ds(i * ZP, ZP), h, :, :], sems_out[b]
        )
    out_flight[0].wait()
    out_flight[1].wait()


@functools.partial(
    pl.kernel,
    mesh=plsc.VectorSubcoreMesh(core_axis_name="c", subcore_axis_name="s"),
    out_type=jax.ShapeDtypeStruct((1, Z, H, W, C), jnp.float32),
    scratch_types=[
        pltpu.VMEM((ZP, W, C), jnp.float32),
        pltpu.VMEM((ZP, W, C), jnp.float32),
        pltpu.VMEM((ZP, W, C), jnp.float32),
        pltpu.VMEM((ZP, W, C), jnp.float32),
        pltpu.SemaphoreType.DMA,
        pltpu.SemaphoreType.DMA,
        pltpu.SemaphoreType.DMA,
        pltpu.SemaphoreType.DMA,
    ],
)
def _fold_sc(x_hbm, out_hbm, bi0, bi1, bo0, bo1, si0, si1, so0, so1):
    _fold_body(x_hbm, out_hbm, bi0, bi1, bo0, bo1, si0, si1, so0, so1)


def kernel(x):
    xt = jnp.transpose(x, (0, 2, 3, 4, 1))
    yt = _fold_sc(xt)
    return jnp.transpose(yt, (0, 4, 1, 2, 3))
